# baseline (speedup 1.0000x reference)
import functools

import jax
import jax.numpy as jnp
from jax import lax
from jax.experimental import pallas as pl
from jax.experimental.pallas import tpu as pltpu
from jax.experimental.pallas import tpu_sc as plsc

N_NODES = 10000
N_PAD = 10240
NC = 2
NS = 16
NW = NC * NS
CH = 128
D = 64
STRIPE = N_PAD // NS
E = 320000
ECH = E // CH
RPT = ECH // NW
EXTRA = ECH - RPT * NW
NCH_MAX = RPT + 1

_mesh = plsc.VectorSubcoreMesh(
    core_axis_name="c", subcore_axis_name="s", num_cores=NC, num_subcores=NS
)

_sc_params = pltpu.CompilerParams(use_tc_tiling_on_sc=False)


def _worker(c, s):
    return c * NS + s


def _n_chunks(wid):
    return jnp.where(wid < EXTRA, RPT + 1, RPT)


def _load_my_indices(idx_hbm, idx_vmem, wid):
    pltpu.sync_copy(idx_hbm.at[pl.ds(wid * RPT, RPT)], idx_vmem.at[pl.ds(0, RPT)])

    @pl.when(wid < EXTRA)
    def _():
        pltpu.sync_copy(
            idx_hbm.at[pl.ds(NW * RPT + wid, 1)], idx_vmem.at[pl.ds(RPT, 1)]
        )


@functools.partial(
    pl.kernel,
    out_type=jax.ShapeDtypeStruct((NC, N_PAD), jnp.float32),
    mesh=_mesh,
    compiler_params=_sc_params,
    scratch_types=[
        pltpu.VMEM((NCH_MAX, 1, CH), jnp.int32),
        pltpu.VMEM((CH,), jnp.float32),
        pltpu.VMEM((STRIPE,), jnp.float32),
        pltpu.VMEM_SHARED((N_PAD,), jnp.float32),
        pltpu.SemaphoreType.DMA,
    ],
)
def _sc_degree(dst_hbm, out_hbm, didx, ones_v, zbuf, deg_sh, sem):
    c = lax.axis_index("c")
    s = lax.axis_index("s")
    wid = _worker(c, s)
    nch = _n_chunks(wid)

    def _fill(i, _):
        zbuf[pl.ds(i * 16, 16)] = jnp.zeros((16,), jnp.float32)
        return 0

    lax.fori_loop(0, STRIPE // 16, _fill, 0)
    for i in range(CH // 16):
        ones_v[pl.ds(i * 16, 16)] = jnp.ones((16,), jnp.float32)
    pltpu.sync_copy(zbuf, deg_sh.at[pl.ds(s * STRIPE, STRIPE)])
    _load_my_indices(dst_hbm, didx, wid)
    plsc.subcore_barrier()

    K = 8

    def _grp(g, _):
        for b in range(K):
            ci = g * K + b

            @pl.when(ci < nch)
            def _():
                pltpu.async_copy(ones_v, deg_sh.at[didx.at[ci, 0]], sem, add=True)

        for b in range(K):
            ci = g * K + b

            @pl.when(ci < nch)
            def _():
                pltpu.make_async_copy(ones_v, deg_sh.at[didx.at[ci, 0]], sem).wait()

        return 0

    lax.fori_loop(0, (NCH_MAX + K - 1) // K, _grp, 0)
    plsc.subcore_barrier()
    pltpu.sync_copy(
        deg_sh.at[pl.ds(s * STRIPE, STRIPE)],
        out_hbm.at[c, pl.ds(s * STRIPE, STRIPE)],
    )


MP_K = 4


@functools.partial(
    pl.kernel,
    out_type=jax.ShapeDtypeStruct((NC, N_PAD, D), jnp.float32),
    mesh=_mesh,
    compiler_params=_sc_params,
    scratch_types=[
        pltpu.VMEM((NCH_MAX, CH), jnp.int32),
        pltpu.VMEM((NCH_MAX, 1, CH), jnp.int32),
        *[pltpu.VMEM((CH, D), jnp.float32) for _ in range(MP_K)],
        pltpu.VMEM((STRIPE // 4, D), jnp.float32),
        pltpu.VMEM_SHARED((N_PAD, D), jnp.float32),
        *[pltpu.SemaphoreType.DMA for _ in range(MP_K)],
    ],
)
def _sc_mp(src_hbm, dst_hbm, feat_hbm, out_hbm, sidx, didx, *rest):
    rows = rest[:MP_K]
    zbuf = rest[MP_K]
    acc_sh = rest[MP_K + 1]
    sems = rest[MP_K + 2 :]
    c = lax.axis_index("c")
    s = lax.axis_index("s")
    wid = _worker(c, s)
    nch = _n_chunks(wid)

    def _fill(i, _):
        for f in range(D // 16):
            zbuf[i, pl.ds(f * 16, 16)] = jnp.zeros((16,), jnp.float32)
        return 0

    lax.fori_loop(0, STRIPE // 4, _fill, 0)
    for t in range(4):
        pltpu.sync_copy(
            zbuf, acc_sh.at[pl.ds(s * STRIPE + t * (STRIPE // 4), STRIPE // 4)]
        )
    _load_my_indices(src_hbm, sidx, wid)
    _load_my_indices(dst_hbm, didx, wid)
    plsc.subcore_barrier()

    for b in range(MP_K):
        pltpu.async_copy(feat_hbm.at[sidx.at[b]], rows[b], sems[b])

    def _grp(g, _):
        for b in range(MP_K):
            ci = g * MP_K + b

            @pl.when(ci < nch)
            def _():
                pltpu.make_async_copy(feat_hbm.at[sidx.at[ci]], rows[b], sems[b]).wait()
                pltpu.sync_copy(rows[b], acc_sh.at[didx.at[ci, 0]], add=True)

                @pl.when(ci + MP_K < nch)
                def _():
                    pltpu.async_copy(feat_hbm.at[sidx.at[ci + MP_K]], rows[b], sems[b])

        return 0

    lax.fori_loop(0, (NCH_MAX + MP_K - 1) // MP_K, _grp, 0)
    plsc.subcore_barrier()
    pltpu.sync_copy(
        acc_sh.at[pl.ds(s * STRIPE, STRIPE)],
        out_hbm.at[c, pl.ds(s * STRIPE, STRIPE)],
    )


DC_K = 3


@functools.partial(
    pl.kernel,
    out_type=jax.ShapeDtypeStruct((ECH, CH), jnp.float32),
    mesh=_mesh,
    compiler_params=pltpu.CompilerParams(
        use_tc_tiling_on_sc=False, needs_layout_passes=False
    ),
    scratch_types=[
        pltpu.VMEM((NCH_MAX, CH), jnp.int32),
        pltpu.VMEM((NCH_MAX, CH), jnp.int32),
        *[pltpu.VMEM((CH, D), jnp.float32) for _ in range(2 * DC_K)],
        pltpu.VMEM((NCH_MAX, CH), jnp.float32),
        *[pltpu.SemaphoreType.DMA for _ in range(2 * DC_K)],
    ],
)
def _sc_decode(src_hbm, dst_hbm, z_hbm, out_hbm, sidx, didx, *rest):
    rows_s = rest[:DC_K]
    rows_d = rest[DC_K : 2 * DC_K]
    lbuf = rest[2 * DC_K]
    sem_s = rest[2 * DC_K + 1 : 2 * DC_K + 1 + DC_K]
    sem_d = rest[2 * DC_K + 1 + DC_K :]
    c = lax.axis_index("c")
    s = lax.axis_index("s")
    wid = _worker(c, s)
    nch = _n_chunks(wid)
    lane = lax.iota(jnp.int32, 16)

    _load_my_indices(src_hbm, sidx, wid)
    _load_my_indices(dst_hbm, didx, wid)

    for b in range(DC_K):
        pltpu.async_copy(z_hbm.at[sidx.at[b]], rows_s[b], sem_s[b])
        pltpu.async_copy(z_hbm.at[didx.at[b]], rows_d[b], sem_d[b])

    def _grp(g, _):
        for b in range(DC_K):
            ci = g * DC_K + b

            @pl.when(ci < nch)
            def _():
                pltpu.make_async_copy(z_hbm.at[sidx.at[ci]], rows_s[b], sem_s[b]).wait()
                pltpu.make_async_copy(z_hbm.at[didx.at[ci]], rows_d[b], sem_d[b]).wait()

                def _group16(gi, _):
                    rowi = gi * 16 + lane
                    accs = [jnp.zeros((16,), jnp.float32) for _ in range(8)]
                    for f in range(D):
                        colf = jnp.full((16,), f, jnp.int32)
                        a = plsc.load_gather(rows_s[b], [rowi, colf])
                        bb = plsc.load_gather(rows_d[b], [rowi, colf])
                        accs[f % 8] = accs[f % 8] + a * bb
                    t0 = (accs[0] + accs[1]) + (accs[2] + accs[3])
                    t1 = (accs[4] + accs[5]) + (accs[6] + accs[7])
                    lbuf[ci, pl.ds(gi * 16, 16)] = t0 + t1
                    return 0

                lax.fori_loop(0, CH // 16, _group16, 0)

                @pl.when(ci + DC_K < nch)
                def _():
                    pltpu.async_copy(z_hbm.at[sidx.at[ci + DC_K]], rows_s[b], sem_s[b])
                    pltpu.async_copy(z_hbm.at[didx.at[ci + DC_K]], rows_d[b], sem_d[b])

        return 0

    lax.fori_loop(0, (NCH_MAX + DC_K - 1) // DC_K, _grp, 0)
    pltpu.sync_copy(lbuf.at[pl.ds(0, RPT)], out_hbm.at[pl.ds(wid * RPT, RPT)])

    @pl.when(wid < EXTRA)
    def _():
        pltpu.sync_copy(lbuf.at[pl.ds(RPT, 1)], out_hbm.at[pl.ds(NW * RPT + wid, 1)])


def _tc_dinv_body(parts_ref, dinv_ref):
    deg = parts_ref[0:1, :] + parts_ref[1:2, :] + 1.0
    dinv_ref[...] = lax.rsqrt(deg)


def _tc_xw_body(x_ref, w1_ref, dinv_ref, xw_ref, xws_ref):
    xw = jnp.dot(x_ref[...], w1_ref[...], preferred_element_type=jnp.float32)
    xw_ref[...] = xw
    xws_ref[...] = xw * dinv_ref[...]


def _tc_layer2_body(acc_ref, xw_ref, dinv_ref, b1_ref, w2_ref, hw_ref, hws_ref):
    accs = acc_ref[0, :N_NODES, :] + acc_ref[1, :N_NODES, :]
    dinv = dinv_ref[...]
    h = jnp.maximum((accs + dinv * xw_ref[...]) * dinv + b1_ref[...], 0.0)
    hw = jnp.dot(h, w2_ref[...], preferred_element_type=jnp.float32)
    hw_ref[...] = hw
    hws_ref[...] = hw * dinv


def _tc_z_body(acc_ref, hw_ref, dinv_ref, b2_ref, z_ref):
    accs = acc_ref[0, :N_NODES, :] + acc_ref[1, :N_NODES, :]
    dinv = dinv_ref[...]
    z_ref[...] = (accs + dinv * hw_ref[...]) * dinv + b2_ref[...]


@jax.jit
def kernel(x, edge_index, W1, b1, W2, b2):
    n = x.shape[0]
    src = edge_index[0].astype(jnp.int32)
    dst = edge_index[1].astype(jnp.int32)
    src2 = src.reshape(ECH, CH)
    dst3 = dst.reshape(ECH, 1, CH)
    dst2 = dst.reshape(ECH, CH)

    deg_parts = _sc_degree(dst3)

    dinv_row = pl.pallas_call(
        _tc_dinv_body,
        out_shape=jax.ShapeDtypeStruct((1, N_PAD), jnp.float32),
    )(deg_parts)
    dinv_col = dinv_row.reshape(N_PAD, 1)[:n]

    xw, xws = pl.pallas_call(
        _tc_xw_body,
        out_shape=(
            jax.ShapeDtypeStruct((n, D), jnp.float32),
            jax.ShapeDtypeStruct((n, D), jnp.float32),
        ),
    )(x, W1, dinv_col)

    acc1 = _sc_mp(src2, dst3, xws)

    hw, hws = pl.pallas_call(
        _tc_layer2_body,
        out_shape=(
            jax.ShapeDtypeStruct((n, D), jnp.float32),
            jax.ShapeDtypeStruct((n, D), jnp.float32),
        ),
    )(acc1, xw, dinv_col, b1, W2)

    acc2 = _sc_mp(src2, dst3, hws)

    z = pl.pallas_call(
        _tc_z_body,
        out_shape=jax.ShapeDtypeStruct((n, D), jnp.float32),
    )(acc2, hw, dinv_col, b2)

    logits = _sc_decode(src2, dst2, z)
    return logits.reshape(E)


# speedup vs baseline: 1.0110x; 1.0110x over previous
import functools

import jax
import jax.numpy as jnp
from jax import lax
from jax.experimental import pallas as pl
from jax.experimental.pallas import tpu as pltpu
from jax.experimental.pallas import tpu_sc as plsc

N_NODES = 10000
N_PAD = 10240
NC = 2
NS = 16
NW = NC * NS
CH = 128
D = 64
STRIPE = N_PAD // NS
E = 320000
ECH = E // CH
RPT = ECH // NW
EXTRA = ECH - RPT * NW
NCH_MAX = RPT + 1

_mesh = plsc.VectorSubcoreMesh(
    core_axis_name="c", subcore_axis_name="s", num_cores=NC, num_subcores=NS
)

_sc_params = pltpu.CompilerParams(use_tc_tiling_on_sc=False)


def _worker(c, s):
    return c * NS + s


def _n_chunks(wid):
    return jnp.where(wid < EXTRA, RPT + 1, RPT)


def _load_my_indices(idx_hbm, idx_vmem, wid):
    pltpu.sync_copy(idx_hbm.at[pl.ds(wid * RPT, RPT)], idx_vmem.at[pl.ds(0, RPT)])

    @pl.when(wid < EXTRA)
    def _():
        pltpu.sync_copy(
            idx_hbm.at[pl.ds(NW * RPT + wid, 1)], idx_vmem.at[pl.ds(RPT, 1)]
        )


@functools.partial(
    pl.kernel,
    out_type=jax.ShapeDtypeStruct((NC, N_PAD), jnp.float32),
    mesh=_mesh,
    compiler_params=_sc_params,
    scratch_types=[
        pltpu.VMEM((NCH_MAX, 1, CH), jnp.int32),
        pltpu.VMEM((CH,), jnp.float32),
        pltpu.VMEM((STRIPE,), jnp.float32),
        pltpu.VMEM_SHARED((N_PAD,), jnp.float32),
        pltpu.SemaphoreType.DMA,
    ],
)
def _sc_degree(dst_hbm, out_hbm, didx, ones_v, zbuf, deg_sh, sem):
    c = lax.axis_index("c")
    s = lax.axis_index("s")
    wid = _worker(c, s)
    nch = _n_chunks(wid)

    def _fill(i, _):
        zbuf[pl.ds(i * 16, 16)] = jnp.zeros((16,), jnp.float32)
        return 0

    lax.fori_loop(0, STRIPE // 16, _fill, 0)
    for i in range(CH // 16):
        ones_v[pl.ds(i * 16, 16)] = jnp.ones((16,), jnp.float32)
    pltpu.sync_copy(zbuf, deg_sh.at[pl.ds(s * STRIPE, STRIPE)])
    _load_my_indices(dst_hbm, didx, wid)
    plsc.subcore_barrier()

    K = 8

    def _grp(g, _):
        for b in range(K):
            ci = g * K + b

            @pl.when(ci < nch)
            def _():
                pltpu.async_copy(ones_v, deg_sh.at[didx.at[ci, 0]], sem, add=True)

        for b in range(K):
            ci = g * K + b

            @pl.when(ci < nch)
            def _():
                pltpu.make_async_copy(ones_v, deg_sh.at[didx.at[ci, 0]], sem).wait()

        return 0

    lax.fori_loop(0, (NCH_MAX + K - 1) // K, _grp, 0)
    plsc.subcore_barrier()
    pltpu.sync_copy(
        deg_sh.at[pl.ds(s * STRIPE, STRIPE)],
        out_hbm.at[c, pl.ds(s * STRIPE, STRIPE)],
    )


MP_K = 4


@functools.partial(
    pl.kernel,
    out_type=jax.ShapeDtypeStruct((NC, N_PAD, D), jnp.float32),
    mesh=_mesh,
    compiler_params=_sc_params,
    scratch_types=[
        pltpu.VMEM((NCH_MAX, CH), jnp.int32),
        pltpu.VMEM((NCH_MAX, 1, CH), jnp.int32),
        *[pltpu.VMEM((CH, D), jnp.float32) for _ in range(MP_K)],
        pltpu.VMEM((STRIPE // 4, D), jnp.float32),
        pltpu.VMEM_SHARED((N_PAD, D), jnp.float32),
        *[pltpu.SemaphoreType.DMA for _ in range(MP_K)],
    ],
)
def _sc_mp(src_hbm, dst_hbm, feat_hbm, out_hbm, sidx, didx, *rest):
    rows = rest[:MP_K]
    zbuf = rest[MP_K]
    acc_sh = rest[MP_K + 1]
    sems = rest[MP_K + 2 :]
    c = lax.axis_index("c")
    s = lax.axis_index("s")
    wid = _worker(c, s)
    nch = _n_chunks(wid)

    def _fill(i, _):
        for f in range(D // 16):
            zbuf[i, pl.ds(f * 16, 16)] = jnp.zeros((16,), jnp.float32)
        return 0

    lax.fori_loop(0, STRIPE // 4, _fill, 0)
    for t in range(4):
        pltpu.sync_copy(
            zbuf, acc_sh.at[pl.ds(s * STRIPE + t * (STRIPE // 4), STRIPE // 4)]
        )
    _load_my_indices(src_hbm, sidx, wid)
    _load_my_indices(dst_hbm, didx, wid)
    plsc.subcore_barrier()

    for b in range(MP_K):
        pltpu.async_copy(feat_hbm.at[sidx.at[b]], rows[b], sems[b])

    def _grp(g, _):
        for b in range(MP_K):
            ci = g * MP_K + b

            @pl.when(ci < nch)
            def _():
                pltpu.make_async_copy(feat_hbm.at[sidx.at[ci]], rows[b], sems[b]).wait()
                pltpu.sync_copy(rows[b], acc_sh.at[didx.at[ci, 0]], add=True)

                @pl.when(ci + MP_K < nch)
                def _():
                    pltpu.async_copy(feat_hbm.at[sidx.at[ci + MP_K]], rows[b], sems[b])

        return 0

    lax.fori_loop(0, (NCH_MAX + MP_K - 1) // MP_K, _grp, 0)
    plsc.subcore_barrier()
    pltpu.sync_copy(
        acc_sh.at[pl.ds(s * STRIPE, STRIPE)],
        out_hbm.at[c, pl.ds(s * STRIPE, STRIPE)],
    )


DC_K = 3


@functools.partial(
    pl.kernel,
    out_type=jax.ShapeDtypeStruct((ECH, CH), jnp.float32),
    mesh=_mesh,
    compiler_params=pltpu.CompilerParams(
        use_tc_tiling_on_sc=False, needs_layout_passes=False
    ),
    scratch_types=[
        pltpu.VMEM((NCH_MAX, CH), jnp.int32),
        pltpu.VMEM((NCH_MAX, CH), jnp.int32),
        *[pltpu.VMEM((CH, D), jnp.float32) for _ in range(2 * DC_K)],
        pltpu.VMEM((NCH_MAX, CH), jnp.float32),
        *[pltpu.SemaphoreType.DMA for _ in range(2 * DC_K)],
    ],
)
def _sc_decode(src_hbm, dst_hbm, z_hbm, out_hbm, sidx, didx, *rest):
    rows_s = rest[:DC_K]
    rows_d = rest[DC_K : 2 * DC_K]
    lbuf = rest[2 * DC_K]
    sem_s = rest[2 * DC_K + 1 : 2 * DC_K + 1 + DC_K]
    sem_d = rest[2 * DC_K + 1 + DC_K :]
    c = lax.axis_index("c")
    s = lax.axis_index("s")
    wid = _worker(c, s)
    nch = _n_chunks(wid)
    lane = lax.iota(jnp.int32, 16)

    _load_my_indices(src_hbm, sidx, wid)
    _load_my_indices(dst_hbm, didx, wid)

    for b in range(DC_K):
        pltpu.async_copy(z_hbm.at[sidx.at[b]], rows_s[b], sem_s[b])
        pltpu.async_copy(z_hbm.at[didx.at[b]], rows_d[b], sem_d[b])

    def _grp(g, _):
        for b in range(DC_K):
            ci = g * DC_K + b

            @pl.when(ci < nch)
            def _():
                pltpu.make_async_copy(z_hbm.at[sidx.at[ci]], rows_s[b], sem_s[b]).wait()
                pltpu.make_async_copy(z_hbm.at[didx.at[ci]], rows_d[b], sem_d[b]).wait()

                def _group16(gi, _):
                    rowi = gi * 16 + lane
                    zero = jnp.zeros((16,), jnp.float32)

                    @plsc.parallel_loop(0, D, 2, unroll=4, carry=(zero, zero))
                    def _facc(f0, carry):
                        a0, a1 = carry
                        c0 = jnp.full((16,), f0, jnp.int32)
                        c1 = jnp.full((16,), f0 + 1, jnp.int32)
                        a0 = a0 + plsc.load_gather(rows_s[b], [rowi, c0]) * plsc.load_gather(rows_d[b], [rowi, c0])
                        a1 = a1 + plsc.load_gather(rows_s[b], [rowi, c1]) * plsc.load_gather(rows_d[b], [rowi, c1])
                        return a0, a1

                    a0, a1 = _facc
                    lbuf[ci, pl.ds(gi * 16, 16)] = a0 + a1
                    return 0

                lax.fori_loop(0, CH // 16, _group16, 0)

                @pl.when(ci + DC_K < nch)
                def _():
                    pltpu.async_copy(z_hbm.at[sidx.at[ci + DC_K]], rows_s[b], sem_s[b])
                    pltpu.async_copy(z_hbm.at[didx.at[ci + DC_K]], rows_d[b], sem_d[b])

        return 0

    lax.fori_loop(0, (NCH_MAX + DC_K - 1) // DC_K, _grp, 0)
    pltpu.sync_copy(lbuf.at[pl.ds(0, RPT)], out_hbm.at[pl.ds(wid * RPT, RPT)])

    @pl.when(wid < EXTRA)
    def _():
        pltpu.sync_copy(lbuf.at[pl.ds(RPT, 1)], out_hbm.at[pl.ds(NW * RPT + wid, 1)])


def _tc_dinv_body(parts_ref, dinv_ref):
    deg = parts_ref[0:1, :] + parts_ref[1:2, :] + 1.0
    dinv_ref[...] = lax.rsqrt(deg)


def _tc_xw_body(x_ref, w1_ref, dinv_ref, xw_ref, xws_ref):
    xw = jnp.dot(x_ref[...], w1_ref[...], preferred_element_type=jnp.float32)
    xw_ref[...] = xw
    xws_ref[...] = xw * dinv_ref[...]


def _tc_layer2_body(acc_ref, xw_ref, dinv_ref, b1_ref, w2_ref, hw_ref, hws_ref):
    accs = acc_ref[0, :N_NODES, :] + acc_ref[1, :N_NODES, :]
    dinv = dinv_ref[...]
    h = jnp.maximum((accs + dinv * xw_ref[...]) * dinv + b1_ref[...], 0.0)
    hw = jnp.dot(h, w2_ref[...], preferred_element_type=jnp.float32)
    hw_ref[...] = hw
    hws_ref[...] = hw * dinv


def _tc_z_body(acc_ref, hw_ref, dinv_ref, b2_ref, z_ref):
    accs = acc_ref[0, :N_NODES, :] + acc_ref[1, :N_NODES, :]
    dinv = dinv_ref[...]
    z_ref[...] = (accs + dinv * hw_ref[...]) * dinv + b2_ref[...]


@jax.jit
def kernel(x, edge_index, W1, b1, W2, b2):
    n = x.shape[0]
    src = edge_index[0].astype(jnp.int32)
    dst = edge_index[1].astype(jnp.int32)
    src2 = src.reshape(ECH, CH)
    dst3 = dst.reshape(ECH, 1, CH)
    dst2 = dst.reshape(ECH, CH)

    deg_parts = _sc_degree(dst3)

    dinv_row = pl.pallas_call(
        _tc_dinv_body,
        out_shape=jax.ShapeDtypeStruct((1, N_PAD), jnp.float32),
    )(deg_parts)
    dinv_col = dinv_row.reshape(N_PAD, 1)[:n]

    xw, xws = pl.pallas_call(
        _tc_xw_body,
        out_shape=(
            jax.ShapeDtypeStruct((n, D), jnp.float32),
            jax.ShapeDtypeStruct((n, D), jnp.float32),
        ),
    )(x, W1, dinv_col)

    acc1 = _sc_mp(src2, dst3, xws)

    hw, hws = pl.pallas_call(
        _tc_layer2_body,
        out_shape=(
            jax.ShapeDtypeStruct((n, D), jnp.float32),
            jax.ShapeDtypeStruct((n, D), jnp.float32),
        ),
    )(acc1, xw, dinv_col, b1, W2)

    acc2 = _sc_mp(src2, dst3, hws)

    z = pl.pallas_call(
        _tc_z_body,
        out_shape=jax.ShapeDtypeStruct((n, D), jnp.float32),
    )(acc2, hw, dinv_col, b2)

    logits = _sc_decode(src2, dst2, z)
    return logits.reshape(E)


# speedup vs baseline: 2.9722x; 2.9399x over previous
import functools

import jax
import jax.numpy as jnp
from jax import lax
from jax.experimental import pallas as pl
from jax.experimental.pallas import tpu as pltpu
from jax.experimental.pallas import tpu_sc as plsc

N_NODES = 10000
N_PAD = 10240
NC = 2
NS = 16
NW = NC * NS
CH = 128
D = 64
STRIPE = N_PAD // NS
E = 320000
ECH = E // CH
RPT = ECH // NW
EXTRA = ECH - RPT * NW
NCH_MAX = RPT + 1

_mesh = plsc.VectorSubcoreMesh(
    core_axis_name="c", subcore_axis_name="s", num_cores=NC, num_subcores=NS
)

_sc_params = pltpu.CompilerParams(use_tc_tiling_on_sc=False)


def _worker(c, s):
    return c * NS + s


def _n_chunks(wid):
    return jnp.where(wid < EXTRA, RPT + 1, RPT)


def _load_my_indices(idx_hbm, idx_vmem, wid):
    pltpu.sync_copy(idx_hbm.at[pl.ds(wid * RPT, RPT)], idx_vmem.at[pl.ds(0, RPT)])

    @pl.when(wid < EXTRA)
    def _():
        pltpu.sync_copy(
            idx_hbm.at[pl.ds(NW * RPT + wid, 1)], idx_vmem.at[pl.ds(RPT, 1)]
        )


@functools.partial(
    pl.kernel,
    out_type=jax.ShapeDtypeStruct((NC, N_PAD), jnp.float32),
    mesh=_mesh,
    compiler_params=_sc_params,
    scratch_types=[
        pltpu.VMEM((NCH_MAX, 1, CH), jnp.int32),
        pltpu.VMEM((CH,), jnp.float32),
        pltpu.VMEM((STRIPE,), jnp.float32),
        pltpu.VMEM_SHARED((N_PAD,), jnp.float32),
        pltpu.SemaphoreType.DMA,
    ],
)
def _sc_degree(dst_hbm, out_hbm, didx, ones_v, zbuf, deg_sh, sem):
    c = lax.axis_index("c")
    s = lax.axis_index("s")
    wid = _worker(c, s)
    nch = _n_chunks(wid)

    def _fill(i, _):
        zbuf[pl.ds(i * 16, 16)] = jnp.zeros((16,), jnp.float32)
        return 0

    lax.fori_loop(0, STRIPE // 16, _fill, 0)
    for i in range(CH // 16):
        ones_v[pl.ds(i * 16, 16)] = jnp.ones((16,), jnp.float32)
    pltpu.sync_copy(zbuf, deg_sh.at[pl.ds(s * STRIPE, STRIPE)])
    _load_my_indices(dst_hbm, didx, wid)
    plsc.subcore_barrier()

    K = 8

    def _grp(g, _):
        for b in range(K):
            ci = g * K + b

            @pl.when(ci < nch)
            def _():
                pltpu.async_copy(ones_v, deg_sh.at[didx.at[ci, 0]], sem, add=True)

        for b in range(K):
            ci = g * K + b

            @pl.when(ci < nch)
            def _():
                pltpu.make_async_copy(ones_v, deg_sh.at[didx.at[ci, 0]], sem).wait()

        return 0

    lax.fori_loop(0, (NCH_MAX + K - 1) // K, _grp, 0)
    plsc.subcore_barrier()
    pltpu.sync_copy(
        deg_sh.at[pl.ds(s * STRIPE, STRIPE)],
        out_hbm.at[c, pl.ds(s * STRIPE, STRIPE)],
    )


MP_K = 4


@functools.partial(
    pl.kernel,
    out_type=jax.ShapeDtypeStruct((NC, N_PAD, D), jnp.float32),
    mesh=_mesh,
    compiler_params=_sc_params,
    scratch_types=[
        pltpu.VMEM((NCH_MAX, CH), jnp.int32),
        pltpu.VMEM((NCH_MAX, 1, CH), jnp.int32),
        *[pltpu.VMEM((CH, D), jnp.float32) for _ in range(MP_K)],
        pltpu.VMEM((STRIPE // 4, D), jnp.float32),
        pltpu.VMEM_SHARED((N_PAD, D), jnp.float32),
        *[pltpu.SemaphoreType.DMA for _ in range(MP_K)],
    ],
)
def _sc_mp(src_hbm, dst_hbm, feat_hbm, out_hbm, sidx, didx, *rest):
    rows = rest[:MP_K]
    zbuf = rest[MP_K]
    acc_sh = rest[MP_K + 1]
    sems = rest[MP_K + 2 :]
    c = lax.axis_index("c")
    s = lax.axis_index("s")
    wid = _worker(c, s)
    nch = _n_chunks(wid)

    def _fill(i, _):
        for f in range(D // 16):
            zbuf[i, pl.ds(f * 16, 16)] = jnp.zeros((16,), jnp.float32)
        return 0

    lax.fori_loop(0, STRIPE // 4, _fill, 0)
    for t in range(4):
        pltpu.sync_copy(
            zbuf, acc_sh.at[pl.ds(s * STRIPE + t * (STRIPE // 4), STRIPE // 4)]
        )
    _load_my_indices(src_hbm, sidx, wid)
    _load_my_indices(dst_hbm, didx, wid)
    plsc.subcore_barrier()

    for b in range(MP_K):
        pltpu.async_copy(feat_hbm.at[sidx.at[b]], rows[b], sems[b])

    def _grp(g, _):
        for b in range(MP_K):
            ci = g * MP_K + b

            @pl.when(ci < nch)
            def _():
                pltpu.make_async_copy(feat_hbm.at[sidx.at[ci]], rows[b], sems[b]).wait()
                pltpu.sync_copy(rows[b], acc_sh.at[didx.at[ci, 0]], add=True)

                @pl.when(ci + MP_K < nch)
                def _():
                    pltpu.async_copy(feat_hbm.at[sidx.at[ci + MP_K]], rows[b], sems[b])

        return 0

    lax.fori_loop(0, (NCH_MAX + MP_K - 1) // MP_K, _grp, 0)
    plsc.subcore_barrier()
    pltpu.sync_copy(
        acc_sh.at[pl.ds(s * STRIPE, STRIPE)],
        out_hbm.at[c, pl.ds(s * STRIPE, STRIPE)],
    )


DC_K = 3


@functools.partial(
    pl.kernel,
    out_type=jax.ShapeDtypeStruct((ECH, CH), jnp.float32),
    mesh=_mesh,
    compiler_params=pltpu.CompilerParams(
        use_tc_tiling_on_sc=False, needs_layout_passes=False
    ),
    scratch_types=[
        pltpu.VMEM((NCH_MAX, CH), jnp.int32),
        pltpu.VMEM((NCH_MAX, CH), jnp.int32),
        *[pltpu.VMEM((CH, D), jnp.float32) for _ in range(2 * DC_K)],
        pltpu.VMEM((NCH_MAX, CH), jnp.float32),
        *[pltpu.SemaphoreType.DMA for _ in range(2 * DC_K)],
    ],
)
def _sc_decode(src_hbm, dst_hbm, z_hbm, out_hbm, sidx, didx, *rest):
    rows_s = rest[:DC_K]
    rows_d = rest[DC_K : 2 * DC_K]
    lbuf = rest[2 * DC_K]
    sem_s = rest[2 * DC_K + 1 : 2 * DC_K + 1 + DC_K]
    sem_d = rest[2 * DC_K + 1 + DC_K :]
    c = lax.axis_index("c")
    s = lax.axis_index("s")
    wid = _worker(c, s)
    nch = _n_chunks(wid)
    lane = lax.iota(jnp.int32, 16)

    _load_my_indices(src_hbm, sidx, wid)
    _load_my_indices(dst_hbm, didx, wid)

    for b in range(DC_K):
        pltpu.async_copy(z_hbm.at[sidx.at[b]], rows_s[b], sem_s[b])
        pltpu.async_copy(z_hbm.at[didx.at[b]], rows_d[b], sem_d[b])

    def _grp(g, _):
        for b in range(DC_K):
            ci = g * DC_K + b

            @pl.when(ci < nch)
            def _():
                pltpu.make_async_copy(z_hbm.at[sidx.at[ci]], rows_s[b], sem_s[b]).wait()
                pltpu.make_async_copy(z_hbm.at[didx.at[ci]], rows_d[b], sem_d[b]).wait()

                def _group16(gi, _):
                    rowi = gi * 16 + lane
                    zero = jnp.zeros((16,), jnp.float32)

                    @plsc.parallel_loop(0, D, 2, unroll=4, carry=(zero, zero))
                    def _facc(f0, carry):
                        a0, a1 = carry
                        c0 = (lane + f0) & (D - 1)
                        c1 = (lane + f0 + 1) & (D - 1)
                        a0 = a0 + plsc.load_gather(rows_s[b], [rowi, c0]) * plsc.load_gather(rows_d[b], [rowi, c0])
                        a1 = a1 + plsc.load_gather(rows_s[b], [rowi, c1]) * plsc.load_gather(rows_d[b], [rowi, c1])
                        return a0, a1

                    a0, a1 = _facc
                    lbuf[ci, pl.ds(gi * 16, 16)] = a0 + a1
                    return 0

                lax.fori_loop(0, CH // 16, _group16, 0)

                @pl.when(ci + DC_K < nch)
                def _():
                    pltpu.async_copy(z_hbm.at[sidx.at[ci + DC_K]], rows_s[b], sem_s[b])
                    pltpu.async_copy(z_hbm.at[didx.at[ci + DC_K]], rows_d[b], sem_d[b])

        return 0

    lax.fori_loop(0, (NCH_MAX + DC_K - 1) // DC_K, _grp, 0)
    pltpu.sync_copy(lbuf.at[pl.ds(0, RPT)], out_hbm.at[pl.ds(wid * RPT, RPT)])

    @pl.when(wid < EXTRA)
    def _():
        pltpu.sync_copy(lbuf.at[pl.ds(RPT, 1)], out_hbm.at[pl.ds(NW * RPT + wid, 1)])


def _tc_dinv_body(parts_ref, dinv_ref):
    deg = parts_ref[0:1, :] + parts_ref[1:2, :] + 1.0
    dinv_ref[...] = lax.rsqrt(deg)


def _tc_xw_body(x_ref, w1_ref, dinv_ref, xw_ref, xws_ref):
    xw = jnp.dot(x_ref[...], w1_ref[...], preferred_element_type=jnp.float32)
    xw_ref[...] = xw
    xws_ref[...] = xw * dinv_ref[...]


def _tc_layer2_body(acc_ref, xw_ref, dinv_ref, b1_ref, w2_ref, hw_ref, hws_ref):
    accs = acc_ref[0, :N_NODES, :] + acc_ref[1, :N_NODES, :]
    dinv = dinv_ref[...]
    h = jnp.maximum((accs + dinv * xw_ref[...]) * dinv + b1_ref[...], 0.0)
    hw = jnp.dot(h, w2_ref[...], preferred_element_type=jnp.float32)
    hw_ref[...] = hw
    hws_ref[...] = hw * dinv


def _tc_z_body(acc_ref, hw_ref, dinv_ref, b2_ref, z_ref):
    accs = acc_ref[0, :N_NODES, :] + acc_ref[1, :N_NODES, :]
    dinv = dinv_ref[...]
    z_ref[...] = (accs + dinv * hw_ref[...]) * dinv + b2_ref[...]


@jax.jit
def kernel(x, edge_index, W1, b1, W2, b2):
    n = x.shape[0]
    src = edge_index[0].astype(jnp.int32)
    dst = edge_index[1].astype(jnp.int32)
    src2 = src.reshape(ECH, CH)
    dst3 = dst.reshape(ECH, 1, CH)
    dst2 = dst.reshape(ECH, CH)

    deg_parts = _sc_degree(dst3)

    dinv_row = pl.pallas_call(
        _tc_dinv_body,
        out_shape=jax.ShapeDtypeStruct((1, N_PAD), jnp.float32),
    )(deg_parts)
    dinv_col = dinv_row.reshape(N_PAD, 1)[:n]

    xw, xws = pl.pallas_call(
        _tc_xw_body,
        out_shape=(
            jax.ShapeDtypeStruct((n, D), jnp.float32),
            jax.ShapeDtypeStruct((n, D), jnp.float32),
        ),
    )(x, W1, dinv_col)

    acc1 = _sc_mp(src2, dst3, xws)

    hw, hws = pl.pallas_call(
        _tc_layer2_body,
        out_shape=(
            jax.ShapeDtypeStruct((n, D), jnp.float32),
            jax.ShapeDtypeStruct((n, D), jnp.float32),
        ),
    )(acc1, xw, dinv_col, b1, W2)

    acc2 = _sc_mp(src2, dst3, hws)

    z = pl.pallas_call(
        _tc_z_body,
        out_shape=jax.ShapeDtypeStruct((n, D), jnp.float32),
    )(acc2, hw, dinv_col, b2)

    logits = _sc_decode(src2, dst2, z)
    return logits.reshape(E)


# speedup vs baseline: 2.9977x; 1.0086x over previous
import functools

import jax
import jax.numpy as jnp
from jax import lax
from jax.experimental import pallas as pl
from jax.experimental.pallas import tpu as pltpu
from jax.experimental.pallas import tpu_sc as plsc

N_NODES = 10000
N_PAD = 10240
NC = 2
NS = 16
NW = NC * NS
CH = 128
D = 64
STRIPE = N_PAD // NS
E = 320000
ECH = E // CH
RPT = ECH // NW
EXTRA = ECH - RPT * NW
NCH_MAX = RPT + 1

_mesh = plsc.VectorSubcoreMesh(
    core_axis_name="c", subcore_axis_name="s", num_cores=NC, num_subcores=NS
)

_sc_params = pltpu.CompilerParams(use_tc_tiling_on_sc=False)


def _worker(c, s):
    return c * NS + s


def _n_chunks(wid):
    return jnp.where(wid < EXTRA, RPT + 1, RPT)


def _load_my_indices(idx_hbm, idx_vmem, wid):
    pltpu.sync_copy(idx_hbm.at[pl.ds(wid * RPT, RPT)], idx_vmem.at[pl.ds(0, RPT)])

    @pl.when(wid < EXTRA)
    def _():
        pltpu.sync_copy(
            idx_hbm.at[pl.ds(NW * RPT + wid, 1)], idx_vmem.at[pl.ds(RPT, 1)]
        )


@functools.partial(
    pl.kernel,
    out_type=jax.ShapeDtypeStruct((NC, N_PAD), jnp.float32),
    mesh=_mesh,
    compiler_params=_sc_params,
    scratch_types=[
        pltpu.VMEM((NCH_MAX, 1, CH), jnp.int32),
        pltpu.VMEM((CH,), jnp.float32),
        pltpu.VMEM((STRIPE,), jnp.float32),
        pltpu.VMEM_SHARED((N_PAD,), jnp.float32),
        pltpu.SemaphoreType.DMA,
    ],
)
def _sc_degree(dst_hbm, out_hbm, didx, ones_v, zbuf, deg_sh, sem):
    c = lax.axis_index("c")
    s = lax.axis_index("s")
    wid = _worker(c, s)
    nch = _n_chunks(wid)

    def _fill(i, _):
        zbuf[pl.ds(i * 16, 16)] = jnp.zeros((16,), jnp.float32)
        return 0

    lax.fori_loop(0, STRIPE // 16, _fill, 0)
    for i in range(CH // 16):
        ones_v[pl.ds(i * 16, 16)] = jnp.ones((16,), jnp.float32)
    pltpu.sync_copy(zbuf, deg_sh.at[pl.ds(s * STRIPE, STRIPE)])
    _load_my_indices(dst_hbm, didx, wid)
    plsc.subcore_barrier()

    K = 8

    def _grp(g, _):
        for b in range(K):
            ci = g * K + b

            @pl.when(ci < nch)
            def _():
                pltpu.async_copy(ones_v, deg_sh.at[didx.at[ci, 0]], sem, add=True)

        for b in range(K):
            ci = g * K + b

            @pl.when(ci < nch)
            def _():
                pltpu.make_async_copy(ones_v, deg_sh.at[didx.at[ci, 0]], sem).wait()

        return 0

    lax.fori_loop(0, (NCH_MAX + K - 1) // K, _grp, 0)
    plsc.subcore_barrier()
    pltpu.sync_copy(
        deg_sh.at[pl.ds(s * STRIPE, STRIPE)],
        out_hbm.at[c, pl.ds(s * STRIPE, STRIPE)],
    )


MP_K = 6


@functools.partial(
    pl.kernel,
    out_type=jax.ShapeDtypeStruct((NC, N_PAD, D), jnp.float32),
    mesh=_mesh,
    compiler_params=_sc_params,
    scratch_types=[
        pltpu.VMEM((NCH_MAX, CH), jnp.int32),
        pltpu.VMEM((NCH_MAX, 1, CH), jnp.int32),
        *[pltpu.VMEM((CH, D), jnp.float32) for _ in range(MP_K)],
        pltpu.VMEM((STRIPE // 4, D), jnp.float32),
        pltpu.VMEM_SHARED((N_PAD, D), jnp.float32),
        *[pltpu.SemaphoreType.DMA for _ in range(MP_K)],
    ],
)
def _sc_mp(src_hbm, dst_hbm, feat_hbm, out_hbm, sidx, didx, *rest):
    rows = rest[:MP_K]
    zbuf = rest[MP_K]
    acc_sh = rest[MP_K + 1]
    sems = rest[MP_K + 2 :]
    c = lax.axis_index("c")
    s = lax.axis_index("s")
    wid = _worker(c, s)
    nch = _n_chunks(wid)

    def _fill(i, _):
        for f in range(D // 16):
            zbuf[i, pl.ds(f * 16, 16)] = jnp.zeros((16,), jnp.float32)
        return 0

    lax.fori_loop(0, STRIPE // 4, _fill, 0)
    for t in range(4):
        pltpu.sync_copy(
            zbuf, acc_sh.at[pl.ds(s * STRIPE + t * (STRIPE // 4), STRIPE // 4)]
        )
    _load_my_indices(src_hbm, sidx, wid)
    _load_my_indices(dst_hbm, didx, wid)
    plsc.subcore_barrier()

    for b in range(MP_K):
        pltpu.async_copy(feat_hbm.at[sidx.at[b]], rows[b], sems[b])

    def _grp(g, _):
        for b in range(MP_K):
            ci = g * MP_K + b

            @pl.when(ci < nch)
            def _():
                pltpu.make_async_copy(feat_hbm.at[sidx.at[ci]], rows[b], sems[b]).wait()
                pltpu.sync_copy(rows[b], acc_sh.at[didx.at[ci, 0]], add=True)

                @pl.when(ci + MP_K < nch)
                def _():
                    pltpu.async_copy(feat_hbm.at[sidx.at[ci + MP_K]], rows[b], sems[b])

        return 0

    lax.fori_loop(0, (NCH_MAX + MP_K - 1) // MP_K, _grp, 0)
    plsc.subcore_barrier()
    pltpu.sync_copy(
        acc_sh.at[pl.ds(s * STRIPE, STRIPE)],
        out_hbm.at[c, pl.ds(s * STRIPE, STRIPE)],
    )


DC_K = 4


@functools.partial(
    pl.kernel,
    out_type=jax.ShapeDtypeStruct((ECH, CH), jnp.float32),
    mesh=_mesh,
    compiler_params=pltpu.CompilerParams(
        use_tc_tiling_on_sc=False, needs_layout_passes=False
    ),
    scratch_types=[
        pltpu.VMEM((NCH_MAX, CH), jnp.int32),
        pltpu.VMEM((NCH_MAX, CH), jnp.int32),
        *[pltpu.VMEM((CH, D), jnp.float32) for _ in range(2 * DC_K)],
        pltpu.VMEM((NCH_MAX, CH), jnp.float32),
        *[pltpu.SemaphoreType.DMA for _ in range(2 * DC_K)],
    ],
)
def _sc_decode(src_hbm, dst_hbm, z_hbm, out_hbm, sidx, didx, *rest):
    rows_s = rest[:DC_K]
    rows_d = rest[DC_K : 2 * DC_K]
    lbuf = rest[2 * DC_K]
    sem_s = rest[2 * DC_K + 1 : 2 * DC_K + 1 + DC_K]
    sem_d = rest[2 * DC_K + 1 + DC_K :]
    c = lax.axis_index("c")
    s = lax.axis_index("s")
    wid = _worker(c, s)
    nch = _n_chunks(wid)
    lane = lax.iota(jnp.int32, 16)

    _load_my_indices(src_hbm, sidx, wid)
    _load_my_indices(dst_hbm, didx, wid)

    for b in range(DC_K):
        pltpu.async_copy(z_hbm.at[sidx.at[b]], rows_s[b], sem_s[b])
        pltpu.async_copy(z_hbm.at[didx.at[b]], rows_d[b], sem_d[b])

    def _grp(g, _):
        for b in range(DC_K):
            ci = g * DC_K + b

            @pl.when(ci < nch)
            def _():
                pltpu.make_async_copy(z_hbm.at[sidx.at[ci]], rows_s[b], sem_s[b]).wait()
                pltpu.make_async_copy(z_hbm.at[didx.at[ci]], rows_d[b], sem_d[b]).wait()

                def _group16(gi, _):
                    rowi = gi * 16 + lane
                    zero = jnp.zeros((16,), jnp.float32)

                    @plsc.parallel_loop(0, D, 2, unroll=4, carry=(zero, zero))
                    def _facc(f0, carry):
                        a0, a1 = carry
                        c0 = (lane + f0) & (D - 1)
                        c1 = (lane + f0 + 1) & (D - 1)
                        a0 = a0 + plsc.load_gather(rows_s[b], [rowi, c0]) * plsc.load_gather(rows_d[b], [rowi, c0])
                        a1 = a1 + plsc.load_gather(rows_s[b], [rowi, c1]) * plsc.load_gather(rows_d[b], [rowi, c1])
                        return a0, a1

                    a0, a1 = _facc
                    lbuf[ci, pl.ds(gi * 16, 16)] = a0 + a1
                    return 0

                lax.fori_loop(0, CH // 16, _group16, 0)

                @pl.when(ci + DC_K < nch)
                def _():
                    pltpu.async_copy(z_hbm.at[sidx.at[ci + DC_K]], rows_s[b], sem_s[b])
                    pltpu.async_copy(z_hbm.at[didx.at[ci + DC_K]], rows_d[b], sem_d[b])

        return 0

    lax.fori_loop(0, (NCH_MAX + DC_K - 1) // DC_K, _grp, 0)
    pltpu.sync_copy(lbuf.at[pl.ds(0, RPT)], out_hbm.at[pl.ds(wid * RPT, RPT)])

    @pl.when(wid < EXTRA)
    def _():
        pltpu.sync_copy(lbuf.at[pl.ds(RPT, 1)], out_hbm.at[pl.ds(NW * RPT + wid, 1)])


def _tc_xw_body(x_ref, w1_ref, parts_ref, xw_ref, xws_ref, dinv_ref):
    deg = parts_ref[0:1, :] + parts_ref[1:2, :] + 1.0
    dinv_col = jnp.reshape(lax.rsqrt(deg), (N_PAD, 1))[:N_NODES]
    dinv_ref[...] = dinv_col
    xw = jnp.dot(x_ref[...], w1_ref[...], preferred_element_type=jnp.float32)
    xw_ref[...] = xw
    xws_ref[...] = xw * dinv_col


def _tc_layer2_body(acc_ref, xw_ref, dinv_ref, b1_ref, w2_ref, hw_ref, hws_ref):
    accs = acc_ref[0, :N_NODES, :] + acc_ref[1, :N_NODES, :]
    dinv = dinv_ref[...]
    h = jnp.maximum((accs + dinv * xw_ref[...]) * dinv + b1_ref[...], 0.0)
    hw = jnp.dot(h, w2_ref[...], preferred_element_type=jnp.float32)
    hw_ref[...] = hw
    hws_ref[...] = hw * dinv


def _tc_z_body(acc_ref, hw_ref, dinv_ref, b2_ref, z_ref):
    accs = acc_ref[0, :N_NODES, :] + acc_ref[1, :N_NODES, :]
    dinv = dinv_ref[...]
    z_ref[...] = (accs + dinv * hw_ref[...]) * dinv + b2_ref[...]


@jax.jit
def kernel(x, edge_index, W1, b1, W2, b2):
    n = x.shape[0]
    src = edge_index[0].astype(jnp.int32)
    dst = edge_index[1].astype(jnp.int32)
    src2 = src.reshape(ECH, CH)
    dst3 = dst.reshape(ECH, 1, CH)
    dst2 = dst.reshape(ECH, CH)

    deg_parts = _sc_degree(dst3)

    xw, xws, dinv_col = pl.pallas_call(
        _tc_xw_body,
        out_shape=(
            jax.ShapeDtypeStruct((n, D), jnp.float32),
            jax.ShapeDtypeStruct((n, D), jnp.float32),
            jax.ShapeDtypeStruct((n, 1), jnp.float32),
        ),
    )(x, W1, deg_parts)

    acc1 = _sc_mp(src2, dst3, xws)

    hw, hws = pl.pallas_call(
        _tc_layer2_body,
        out_shape=(
            jax.ShapeDtypeStruct((n, D), jnp.float32),
            jax.ShapeDtypeStruct((n, D), jnp.float32),
        ),
    )(acc1, xw, dinv_col, b1, W2)

    acc2 = _sc_mp(src2, dst3, hws)

    z = pl.pallas_call(
        _tc_z_body,
        out_shape=jax.ShapeDtypeStruct((n, D), jnp.float32),
    )(acc2, hw, dinv_col, b2)

    logits = _sc_decode(src2, dst2, z)
    return logits.reshape(E)


# speedup vs baseline: 3.0145x; 1.0056x over previous
import functools

import jax
import jax.numpy as jnp
from jax import lax
from jax.experimental import pallas as pl
from jax.experimental.pallas import tpu as pltpu
from jax.experimental.pallas import tpu_sc as plsc

N_NODES = 10000
N_PAD = 10240
NC = 2
NS = 16
NW = NC * NS
CH = 128
D = 64
STRIPE = N_PAD // NS
E = 320000
ECH = E // CH
RPT = ECH // NW
EXTRA = ECH - RPT * NW
NCH_MAX = RPT + 1

_mesh = plsc.VectorSubcoreMesh(
    core_axis_name="c", subcore_axis_name="s", num_cores=NC, num_subcores=NS
)

_sc_params = pltpu.CompilerParams(use_tc_tiling_on_sc=False)


def _worker(c, s):
    return c * NS + s


def _n_chunks(wid):
    return jnp.where(wid < EXTRA, RPT + 1, RPT)


def _load_my_indices(idx_hbm, idx_vmem, wid):
    pltpu.sync_copy(idx_hbm.at[pl.ds(wid * RPT, RPT)], idx_vmem.at[pl.ds(0, RPT)])

    @pl.when(wid < EXTRA)
    def _():
        pltpu.sync_copy(
            idx_hbm.at[pl.ds(NW * RPT + wid, 1)], idx_vmem.at[pl.ds(RPT, 1)]
        )


@functools.partial(
    pl.kernel,
    out_type=jax.ShapeDtypeStruct((NC, N_PAD), jnp.float32),
    mesh=_mesh,
    compiler_params=_sc_params,
    scratch_types=[
        pltpu.VMEM((NCH_MAX, 1, CH), jnp.int32),
        pltpu.VMEM((CH,), jnp.float32),
        pltpu.VMEM((STRIPE,), jnp.float32),
        pltpu.VMEM_SHARED((N_PAD,), jnp.float32),
        pltpu.SemaphoreType.DMA,
    ],
)
def _sc_degree(dst_hbm, out_hbm, didx, ones_v, zbuf, deg_sh, sem):
    c = lax.axis_index("c")
    s = lax.axis_index("s")
    wid = _worker(c, s)
    nch = _n_chunks(wid)

    def _fill(i, _):
        zbuf[pl.ds(i * 16, 16)] = jnp.zeros((16,), jnp.float32)
        return 0

    lax.fori_loop(0, STRIPE // 16, _fill, 0)
    for i in range(CH // 16):
        ones_v[pl.ds(i * 16, 16)] = jnp.ones((16,), jnp.float32)
    pltpu.sync_copy(zbuf, deg_sh.at[pl.ds(s * STRIPE, STRIPE)])
    _load_my_indices(dst_hbm, didx, wid)
    plsc.subcore_barrier()

    K = 8

    def _grp(g, _):
        for b in range(K):
            ci = g * K + b

            @pl.when(ci < nch)
            def _():
                pltpu.async_copy(ones_v, deg_sh.at[didx.at[ci, 0]], sem, add=True)

        for b in range(K):
            ci = g * K + b

            @pl.when(ci < nch)
            def _():
                pltpu.make_async_copy(ones_v, deg_sh.at[didx.at[ci, 0]], sem).wait()

        return 0

    lax.fori_loop(0, (NCH_MAX + K - 1) // K, _grp, 0)
    plsc.subcore_barrier()
    pltpu.sync_copy(
        deg_sh.at[pl.ds(s * STRIPE, STRIPE)],
        out_hbm.at[c, pl.ds(s * STRIPE, STRIPE)],
    )


MP_K = 6


@functools.partial(
    pl.kernel,
    out_type=jax.ShapeDtypeStruct((NC, N_PAD, D), jnp.float32),
    mesh=_mesh,
    compiler_params=_sc_params,
    scratch_types=[
        pltpu.VMEM((NCH_MAX, CH), jnp.int32),
        pltpu.VMEM((NCH_MAX, 1, CH), jnp.int32),
        *[pltpu.VMEM((CH, D), jnp.float32) for _ in range(MP_K)],
        pltpu.VMEM((STRIPE // 4, D), jnp.float32),
        pltpu.VMEM_SHARED((N_PAD, D), jnp.float32),
        *[pltpu.SemaphoreType.DMA for _ in range(MP_K)],
    ],
)
def _sc_mp(src_hbm, dst_hbm, feat_hbm, out_hbm, sidx, didx, *rest):
    rows = rest[:MP_K]
    zbuf = rest[MP_K]
    acc_sh = rest[MP_K + 1]
    sems = rest[MP_K + 2 :]
    c = lax.axis_index("c")
    s = lax.axis_index("s")
    wid = _worker(c, s)
    nch = _n_chunks(wid)

    def _fill(i, _):
        for f in range(D // 16):
            zbuf[i, pl.ds(f * 16, 16)] = jnp.zeros((16,), jnp.float32)
        return 0

    lax.fori_loop(0, STRIPE // 4, _fill, 0)
    for t in range(4):
        pltpu.sync_copy(
            zbuf, acc_sh.at[pl.ds(s * STRIPE + t * (STRIPE // 4), STRIPE // 4)]
        )
    _load_my_indices(src_hbm, sidx, wid)
    _load_my_indices(dst_hbm, didx, wid)
    plsc.subcore_barrier()

    for b in range(MP_K):
        pltpu.async_copy(feat_hbm.at[sidx.at[b]], rows[b], sems[b])

    def _grp(g, _):
        for b in range(MP_K):
            ci = g * MP_K + b

            @pl.when(ci < nch)
            def _():
                pltpu.make_async_copy(feat_hbm.at[sidx.at[ci]], rows[b], sems[b]).wait()
                pltpu.sync_copy(rows[b], acc_sh.at[didx.at[ci, 0]], add=True)

                @pl.when(ci + MP_K < nch)
                def _():
                    pltpu.async_copy(feat_hbm.at[sidx.at[ci + MP_K]], rows[b], sems[b])

        return 0

    lax.fori_loop(0, (NCH_MAX + MP_K - 1) // MP_K, _grp, 0)
    plsc.subcore_barrier()
    pltpu.sync_copy(
        acc_sh.at[pl.ds(s * STRIPE, STRIPE)],
        out_hbm.at[c, pl.ds(s * STRIPE, STRIPE)],
    )


DC_K = 4


@functools.partial(
    pl.kernel,
    out_type=jax.ShapeDtypeStruct((ECH, CH), jnp.float32),
    mesh=_mesh,
    compiler_params=pltpu.CompilerParams(
        use_tc_tiling_on_sc=False, needs_layout_passes=False
    ),
    scratch_types=[
        pltpu.VMEM((NCH_MAX, CH), jnp.int32),
        pltpu.VMEM((NCH_MAX, CH), jnp.int32),
        *[pltpu.VMEM((CH, D // 2), jnp.int32) for _ in range(2 * DC_K)],
        pltpu.VMEM((NCH_MAX, CH), jnp.float32),
        *[pltpu.SemaphoreType.DMA for _ in range(2 * DC_K)],
    ],
)
def _sc_decode(src_hbm, dst_hbm, z_hbm, out_hbm, sidx, didx, *rest):
    rows_s = rest[:DC_K]
    rows_d = rest[DC_K : 2 * DC_K]
    lbuf = rest[2 * DC_K]
    sem_s = rest[2 * DC_K + 1 : 2 * DC_K + 1 + DC_K]
    sem_d = rest[2 * DC_K + 1 + DC_K :]
    c = lax.axis_index("c")
    s = lax.axis_index("s")
    wid = _worker(c, s)
    nch = _n_chunks(wid)
    lane = lax.iota(jnp.int32, 16)

    _load_my_indices(src_hbm, sidx, wid)
    _load_my_indices(dst_hbm, didx, wid)

    for b in range(DC_K):
        pltpu.async_copy(z_hbm.at[sidx.at[b]], rows_s[b], sem_s[b])
        pltpu.async_copy(z_hbm.at[didx.at[b]], rows_d[b], sem_d[b])

    def _grp(g, _):
        for b in range(DC_K):
            ci = g * DC_K + b

            @pl.when(ci < nch)
            def _():
                pltpu.make_async_copy(z_hbm.at[sidx.at[ci]], rows_s[b], sem_s[b]).wait()
                pltpu.make_async_copy(z_hbm.at[didx.at[ci]], rows_d[b], sem_d[b]).wait()

                def _group16(gi, _):
                    rowi = gi * 16 + lane
                    zero = jnp.zeros((16,), jnp.float32)

                    @plsc.parallel_loop(0, D // 2, 1, unroll=4, carry=(zero, zero))
                    def _facc(w0, carry):
                        a0, a1 = carry
                        c0 = (lane + w0) & (D // 2 - 1)
                        sw = plsc.load_gather(rows_s[b], [rowi, c0])
                        dw = plsc.load_gather(rows_d[b], [rowi, c0])
                        se, so = plsc.unpack(plsc.bitcast(sw, jnp.bfloat16), format=plsc.PackFormat.INTERLEAVED)
                        de, do_ = plsc.unpack(plsc.bitcast(dw, jnp.bfloat16), format=plsc.PackFormat.INTERLEAVED)
                        a0 = a0 + se * de
                        a1 = a1 + so * do_
                        return a0, a1

                    a0, a1 = _facc
                    lbuf[ci, pl.ds(gi * 16, 16)] = a0 + a1
                    return 0

                lax.fori_loop(0, CH // 16, _group16, 0)

                @pl.when(ci + DC_K < nch)
                def _():
                    pltpu.async_copy(z_hbm.at[sidx.at[ci + DC_K]], rows_s[b], sem_s[b])
                    pltpu.async_copy(z_hbm.at[didx.at[ci + DC_K]], rows_d[b], sem_d[b])

        return 0

    lax.fori_loop(0, (NCH_MAX + DC_K - 1) // DC_K, _grp, 0)
    pltpu.sync_copy(lbuf.at[pl.ds(0, RPT)], out_hbm.at[pl.ds(wid * RPT, RPT)])

    @pl.when(wid < EXTRA)
    def _():
        pltpu.sync_copy(lbuf.at[pl.ds(RPT, 1)], out_hbm.at[pl.ds(NW * RPT + wid, 1)])


def _tc_xw_body(x_ref, w1_ref, parts_ref, xw_ref, xws_ref, dinv_ref):
    deg = parts_ref[0:1, :] + parts_ref[1:2, :] + 1.0
    dinv_col = jnp.reshape(lax.rsqrt(deg), (N_PAD, 1))[:N_NODES]
    dinv_ref[...] = dinv_col
    xw = jnp.dot(x_ref[...], w1_ref[...], preferred_element_type=jnp.float32)
    xw_ref[...] = xw
    xws_ref[...] = xw * dinv_col


def _tc_layer2_body(acc_ref, xw_ref, dinv_ref, b1_ref, w2_ref, hw_ref, hws_ref):
    accs = acc_ref[0, :N_NODES, :] + acc_ref[1, :N_NODES, :]
    dinv = dinv_ref[...]
    h = jnp.maximum((accs + dinv * xw_ref[...]) * dinv + b1_ref[...], 0.0)
    hw = jnp.dot(h, w2_ref[...], preferred_element_type=jnp.float32)
    hw_ref[...] = hw
    hws_ref[...] = hw * dinv


def _tc_z_body(acc_ref, hw_ref, dinv_ref, b2_ref, z_ref):
    accs = acc_ref[0, :N_NODES, :] + acc_ref[1, :N_NODES, :]
    dinv = dinv_ref[...]
    z = (accs + dinv * hw_ref[...]) * dinv + b2_ref[...]
    z_ref[...] = z.astype(jnp.bfloat16)


@jax.jit
def kernel(x, edge_index, W1, b1, W2, b2):
    n = x.shape[0]
    src = edge_index[0].astype(jnp.int32)
    dst = edge_index[1].astype(jnp.int32)
    src2 = src.reshape(ECH, CH)
    dst3 = dst.reshape(ECH, 1, CH)
    dst2 = dst.reshape(ECH, CH)

    deg_parts = _sc_degree(dst3)

    xw, xws, dinv_col = pl.pallas_call(
        _tc_xw_body,
        out_shape=(
            jax.ShapeDtypeStruct((n, D), jnp.float32),
            jax.ShapeDtypeStruct((n, D), jnp.float32),
            jax.ShapeDtypeStruct((n, 1), jnp.float32),
        ),
    )(x, W1, deg_parts)

    acc1 = _sc_mp(src2, dst3, xws)

    hw, hws = pl.pallas_call(
        _tc_layer2_body,
        out_shape=(
            jax.ShapeDtypeStruct((n, D), jnp.float32),
            jax.ShapeDtypeStruct((n, D), jnp.float32),
        ),
    )(acc1, xw, dinv_col, b1, W2)

    acc2 = _sc_mp(src2, dst3, hws)

    z = pl.pallas_call(
        _tc_z_body,
        out_shape=jax.ShapeDtypeStruct((n, D), jnp.bfloat16),
    )(acc2, hw, dinv_col, b2)
    z32 = jax.lax.bitcast_convert_type(z.reshape(n, D // 2, 2), jnp.int32)

    logits = _sc_decode(src2, dst2, z32)
    return logits.reshape(E)
